# baseline (speedup 1.0000x reference)
import jax
import jax.numpy as jnp
from jax.experimental import pallas as pl

_NUM_GRAPHS = 16
_N_NODES = 100000
_IDS_ROWS = 782
_IDS_PAD = _IDS_ROWS * 128 - _N_NODES
_A_ROWS = 12500


def _hist_copy_kernel(ids_ref, a_ref, out_ref, hist_ref):
    ids = ids_ref[...]
    bins = jax.lax.broadcasted_iota(jnp.int32, (_NUM_GRAPHS, 1, 1), 0)
    one_hot = (ids[None, :, :] == bins).astype(jnp.int32)
    partial = jnp.sum(one_hot, axis=1)
    counts = jnp.sum(partial, axis=1, keepdims=True)
    row = jax.lax.broadcasted_iota(jnp.int32, (_NUM_GRAPHS, _NUM_GRAPHS), 0)
    col = jax.lax.broadcasted_iota(jnp.int32, (_NUM_GRAPHS, _NUM_GRAPHS), 1)
    tri = (row >= col).astype(jnp.float32)
    csum = jax.lax.dot(tri, counts.astype(jnp.float32)).astype(jnp.int32)
    hist_ref[...] = jnp.broadcast_to(csum, (_NUM_GRAPHS, 128))
    out_ref[...] = a_ref[...]


def kernel(X, A, I):
    ids = I.astype(jnp.int32)
    ids = jnp.concatenate(
        [ids, jnp.full((_IDS_PAD,), _NUM_GRAPHS, jnp.int32)]
    ).reshape(_IDS_ROWS, 128)
    a2 = A.reshape(_A_ROWS, 128)
    out, _hist = pl.pallas_call(
        _hist_copy_kernel,
        out_shape=(
            jax.ShapeDtypeStruct((_A_ROWS, 128), A.dtype),
            jax.ShapeDtypeStruct((_NUM_GRAPHS, 128), jnp.int32),
        ),
    )(ids, a2)
    return out.reshape(A.shape)
